# baseline (speedup 1.0000x reference)
import functools

import jax
import jax.numpy as jnp
from jax.experimental import pallas as pl
from jax.experimental.pallas import tpu as pltpu

_K = 8
_SLOTS = 16


def _se_ring_kernel(x_hbm, w1_ref, w2_ref, o_hbm, buf, gates, in_sem, out_sem,
                    *, n):
    j = pl.program_id(0)

    def start_in(idx, slot):
        pltpu.make_async_copy(x_hbm.at[idx], buf.at[slot],
                              in_sem.at[slot]).start()

    def wait_in(slot):
        pltpu.make_async_copy(buf.at[slot], buf.at[slot],
                              in_sem.at[slot]).wait()

    def start_out(idx, slot):
        pltpu.make_async_copy(buf.at[slot], o_hbm.at[idx],
                              out_sem.at[slot]).start()

    def wait_out(slot):
        pltpu.make_async_copy(buf.at[slot], buf.at[slot],
                              out_sem.at[slot]).wait()

    @pl.when(j == 0)
    def _():
        for k in range(min(_K, n)):
            start_in(k, k)

    @pl.when(j >= 1)
    def _():
        pslot = jax.lax.rem(j - 1, _SLOTS)
        buf[pslot] = buf[pslot] * gates[pslot].astype(buf.dtype)
        start_out(j - 1, pslot)

    @pl.when(j < n)
    def _():
        slot = jax.lax.rem(j, _SLOTS)
        wait_in(slot)
        y = jnp.sum(buf[slot].astype(jnp.float32), axis=-1, keepdims=True)
        h = jax.lax.dot_general(w1_ref[...], y, (((1,), (0,)), ((), ())),
                                preferred_element_type=jnp.float32)
        h = jnp.maximum(h, 0.0)
        gates[slot] = jax.nn.sigmoid(
            jax.lax.dot_general(w2_ref[...], h, (((1,), (0,)), ((), ())),
                                preferred_element_type=jnp.float32))

    @pl.when(j + _K < n)
    def _():
        nslot = jax.lax.rem(j + _K, _SLOTS)

        @pl.when(j >= _SLOTS - _K)
        def _():
            wait_out(nslot)

        start_in(j + _K, nslot)

    @pl.when(j == n)
    def _():
        for t in range(max(0, n - _SLOTS), n):
            wait_out(t % _SLOTS)


def kernel(x_nchw, w1, w2):
    B, C, H, W = x_nchw.shape
    Cr = w1.shape[0]
    HW = H * W
    dtype = x_nchw.dtype

    x3 = x_nchw.reshape(B, C, HW)
    w1f = (w1 * (1.0 / float(HW))).astype(jnp.float32)
    w2f = w2.astype(jnp.float32)

    out3 = pl.pallas_call(
        functools.partial(_se_ring_kernel, n=B),
        out_shape=jax.ShapeDtypeStruct((B, C, HW), dtype),
        grid_spec=pltpu.PrefetchScalarGridSpec(
            num_scalar_prefetch=0,
            grid=(B + 1,),
            in_specs=[
                pl.BlockSpec(memory_space=pl.ANY),
                pl.BlockSpec((Cr, C), lambda i: (0, 0)),
                pl.BlockSpec((C, Cr), lambda i: (0, 0)),
            ],
            out_specs=pl.BlockSpec(memory_space=pl.ANY),
            scratch_shapes=[
                pltpu.VMEM((_SLOTS, C, HW), dtype),
                pltpu.VMEM((_SLOTS, C, 1), jnp.float32),
                pltpu.SemaphoreType.DMA((_SLOTS,)),
                pltpu.SemaphoreType.DMA((_SLOTS,)),
            ],
        ),
        compiler_params=pltpu.CompilerParams(
            dimension_semantics=("arbitrary",),
            vmem_limit_bytes=56 << 20,
        ),
        cost_estimate=pl.CostEstimate(
            flops=int(3 * B * C * HW + 4 * B * C * Cr),
            transcendentals=int(B * C),
            bytes_accessed=int(2 * B * C * HW * jnp.dtype(dtype).itemsize),
        ),
    )(x3, w1f, w2f)

    return out3.reshape(B, C, H, W)


# speedup vs baseline: 1.0026x; 1.0026x over previous
import functools

import jax
import jax.numpy as jnp
from jax.experimental import pallas as pl
from jax.experimental.pallas import tpu as pltpu

_K = 6
_SLOTS = 12
_Q = 4


def _se_ring_kernel(x_hbm, w1_ref, w2_ref, o_hbm, buf, gates, in_sem, out_sem,
                    *, n):
    j = pl.program_id(0)

    def start_in(idx, slot):
        pltpu.make_async_copy(x_hbm.at[idx], buf.at[slot],
                              in_sem.at[slot]).start()

    def wait_in(slot):
        pltpu.make_async_copy(buf.at[slot], buf.at[slot],
                              in_sem.at[slot]).wait()

    cq = buf.shape[1] // _Q

    def start_out(idx, slot, q):
        sl = pl.ds(q * cq, cq)
        pltpu.make_async_copy(buf.at[slot, sl], o_hbm.at[idx, sl],
                              out_sem.at[slot, q]).start()

    def wait_out(slot, q):
        pltpu.make_async_copy(buf.at[slot, pl.ds(0, cq)],
                              buf.at[slot, pl.ds(0, cq)],
                              out_sem.at[slot, q]).wait()

    @pl.when(j == 0)
    def _():
        for k in range(min(_K, n)):
            start_in(k, k)

    @pl.when(j >= 1)
    def _():
        pslot = jax.lax.rem(j - 1, _SLOTS)
        for q in range(_Q):
            sl = slice(q * cq, (q + 1) * cq)
            buf[pslot, sl] = buf[pslot, sl] * gates[pslot, sl].astype(buf.dtype)
            start_out(j - 1, pslot, q)

    @pl.when(j < n)
    def _():
        slot = jax.lax.rem(j, _SLOTS)
        wait_in(slot)
        y = jnp.sum(buf[slot].astype(jnp.float32), axis=-1, keepdims=True)
        h = jax.lax.dot_general(w1_ref[...], y, (((1,), (0,)), ((), ())),
                                preferred_element_type=jnp.float32)
        h = jnp.maximum(h, 0.0)
        gates[slot] = jax.nn.sigmoid(
            jax.lax.dot_general(w2_ref[...], h, (((1,), (0,)), ((), ())),
                                preferred_element_type=jnp.float32))

    @pl.when(j + _K < n)
    def _():
        nslot = jax.lax.rem(j + _K, _SLOTS)

        @pl.when(j >= _SLOTS - _K)
        def _():
            for q in range(_Q):
                wait_out(nslot, q)

        start_in(j + _K, nslot)

    @pl.when(j == n)
    def _():
        for t in range(max(0, n - _SLOTS), n):
            for q in range(_Q):
                wait_out(t % _SLOTS, q)


def kernel(x_nchw, w1, w2):
    B, C, H, W = x_nchw.shape
    Cr = w1.shape[0]
    HW = H * W
    dtype = x_nchw.dtype

    x3 = x_nchw.reshape(B, C, HW)
    w1f = (w1 * (1.0 / float(HW))).astype(jnp.float32)
    w2f = w2.astype(jnp.float32)

    out3 = pl.pallas_call(
        functools.partial(_se_ring_kernel, n=B),
        out_shape=jax.ShapeDtypeStruct((B, C, HW), dtype),
        grid_spec=pltpu.PrefetchScalarGridSpec(
            num_scalar_prefetch=0,
            grid=(B + 1,),
            in_specs=[
                pl.BlockSpec(memory_space=pl.ANY),
                pl.BlockSpec((Cr, C), lambda i: (0, 0)),
                pl.BlockSpec((C, Cr), lambda i: (0, 0)),
            ],
            out_specs=pl.BlockSpec(memory_space=pl.ANY),
            scratch_shapes=[
                pltpu.VMEM((_SLOTS, C, HW), dtype),
                pltpu.VMEM((_SLOTS, C, 1), jnp.float32),
                pltpu.SemaphoreType.DMA((_SLOTS,)),
                pltpu.SemaphoreType.DMA((_SLOTS, _Q)),
            ],
        ),
        compiler_params=pltpu.CompilerParams(
            dimension_semantics=("arbitrary",),
            vmem_limit_bytes=56 << 20,
        ),
        cost_estimate=pl.CostEstimate(
            flops=int(3 * B * C * HW + 4 * B * C * Cr),
            transcendentals=int(B * C),
            bytes_accessed=int(2 * B * C * HW * jnp.dtype(dtype).itemsize),
        ),
    )(x3, w1f, w2f)

    return out3.reshape(B, C, H, W)
